# baseline (speedup 1.0000x reference)
import functools

import jax
import jax.numpy as jnp
from jax import lax
from jax.experimental import pallas as pl
from jax.experimental.pallas import tpu as pltpu
from jax.experimental.pallas import tpu_sc as plsc

N = 10000
E = 160000
D = 256
H = 128
NC = 2
NS = 16
NT0 = 624
NTL = N - NT0 * (NS - 1)

KA = 128
NBA = 80
ETILE = KA * NBA
EPAD = ETILE * NS
NSLOT = 2
NACC = N + 8

KH = 128
ETH = EPAD // (NC * NS)
NBH = ETH // KH

_MESH = plsc.VectorSubcoreMesh(core_axis_name="c", subcore_axis_name="s")


def _per_tile_rows(s, do):
    @pl.when(s < NS - 1)
    def _():
        do(s * NT0, NT0)

    @pl.when(s == NS - 1)
    def _():
        do((NS - 1) * NT0, NTL)


def _hist_body(dst_hbm, zeros_hbm, ones_hbm, out_hbm, onesv, dstring, cnt_sh,
               *sems):
    dsem = sems[:NSLOT]
    ssem = sems[NSLOT:]
    c = lax.axis_index("c")
    s = lax.axis_index("s")
    _per_tile_rows(s, lambda r, n: pltpu.sync_copy(
        zeros_hbm.at[pl.ds(r, n)], cnt_sh.at[pl.ds(r, n)]))
    pltpu.sync_copy(ones_hbm, onesv)
    plsc.subcore_barrier()
    base0 = (s * NC + c) * ETH

    def start_d(b, i):
        pltpu.async_copy(dst_hbm.at[pl.ds(base0 + i * KH, KH)], dstring.at[b],
                         dsem[b])

    def wait_d(b, i):
        pltpu.make_async_copy(dst_hbm.at[pl.ds(base0 + i * KH, KH)],
                              dstring.at[b], dsem[b]).wait()

    def start_s(b):
        pltpu.async_copy(onesv, cnt_sh.at[dstring.at[b]], ssem[b], add=True)

    def wait_s(b):
        pltpu.make_async_copy(onesv, cnt_sh.at[dstring.at[b]], ssem[b]).wait()

    for b in range(NSLOT):
        start_d(b, b)

    def outer(g, carry):
        for b in range(NSLOT):
            i = g * NSLOT + b
            wait_d(b, i)
            start_s(b)
            wait_s(b)

            @pl.when(i + NSLOT < NBH)
            def _():
                start_d(b, i + NSLOT)
        return carry

    lax.fori_loop(0, NBH // NSLOT, outer, 0)
    plsc.subcore_barrier()
    _per_tile_rows(s, lambda r, n: pltpu.sync_copy(
        cnt_sh.at[pl.ds(r, n)], out_hbm.at[c, pl.ds(r, n)]))


def _sc_hist(dstp_flat, zeros, ones):
    return pl.kernel(
        _hist_body,
        out_type=jax.ShapeDtypeStruct((NC, N, 16), jnp.float32),
        mesh=_MESH,
        scratch_types=[
            pltpu.VMEM((KH, 16), jnp.float32),
            pltpu.VMEM((NSLOT, KH), jnp.int32),
            pltpu.VMEM_SHARED((NACC, 16), jnp.float32),
        ] + [pltpu.SemaphoreType.DMA] * (2 * NSLOT),
    )(dstp_flat, zeros, ones)


def _agg_body(hp_hbm, src2_hbm, dst_hbm, out_hbm, src_all, dstring, rows,
              acc_sh, *sems):
    gsem = sems[:NSLOT]
    ssem = sems[NSLOT:2 * NSLOT]
    dsem = sems[2 * NSLOT:]
    c = lax.axis_index("c")
    s = lax.axis_index("s")
    pltpu.sync_copy(src2_hbm.at[pl.ds(c * EPAD + s * ETILE, ETILE)], src_all)
    _per_tile_rows(s, lambda r, n: pltpu.sync_copy(
        hp_hbm.at[pl.ds(c * N + r, n)], acc_sh.at[pl.ds(r, n)]))
    plsc.subcore_barrier()
    ebase = s * ETILE

    def start_d(b, i):
        pltpu.async_copy(dst_hbm.at[pl.ds(ebase + i * KA, KA)], dstring.at[b],
                         dsem[b])

    def wait_d(b, i):
        pltpu.make_async_copy(dst_hbm.at[pl.ds(ebase + i * KA, KA)],
                              dstring.at[b], dsem[b]).wait()

    def start_g(b, i):
        pltpu.async_copy(hp_hbm.at[src_all.at[pl.ds(i * KA, KA)]], rows.at[b],
                         gsem[b])

    def wait_g(b, i):
        pltpu.make_async_copy(hp_hbm.at[src_all.at[pl.ds(i * KA, KA)]],
                              rows.at[b], gsem[b]).wait()

    def start_s(b, i):
        pltpu.async_copy(rows.at[b], acc_sh.at[dstring.at[b]], ssem[b],
                         add=True)

    def wait_s(b, i):
        pltpu.make_async_copy(rows.at[b], acc_sh.at[dstring.at[b]],
                              ssem[b]).wait()

    for b in range(NSLOT):
        start_d(b, b)
        start_g(b, b)

    def outer(g, carry):
        for b in range(NSLOT):
            i = g * NSLOT + b
            wait_d(b, i)
            wait_g(b, i)
            start_s(b, i)
            wait_s(b, i)

            @pl.when(i + NSLOT < NBA)
            def _():
                start_d(b, i + NSLOT)
                start_g(b, i + NSLOT)
        return carry

    lax.fori_loop(0, NBA // NSLOT, outer, 0)
    plsc.subcore_barrier()
    _per_tile_rows(s, lambda r, n: pltpu.sync_copy(
        acc_sh.at[pl.ds(r, n)], out_hbm.at[c, pl.ds(r, n)]))


def _sc_agg(hp_flat, src2, dst):
    return pl.kernel(
        _agg_body,
        out_type=jax.ShapeDtypeStruct((NC, N, H), jnp.float32),
        mesh=_MESH,
        scratch_types=[
            pltpu.VMEM((ETILE,), jnp.int32),
            pltpu.VMEM((NSLOT, KA), jnp.int32),
            pltpu.VMEM((NSLOT, KA, H), jnp.float32),
            pltpu.VMEM_SHARED((NACC, H), jnp.float32),
        ] + [pltpu.SemaphoreType.DMA] * (3 * NSLOT),
    )(hp_flat, src2, dst)


BN = 1000


def _preA_body(x_ref, w_ref, h_ref):
    h_ref[...] = jnp.dot(x_ref[...], w_ref[...],
                         preferred_element_type=jnp.float32)


def _tc_preA(x, W1):
    return pl.pallas_call(
        _preA_body,
        grid=(N // BN,),
        in_specs=[
            pl.BlockSpec((BN, D), lambda i: (i, 0)),
            pl.BlockSpec((D, D), lambda i: (0, 0)),
        ],
        out_specs=pl.BlockSpec((BN, D), lambda i: (i, 0)),
        out_shape=jax.ShapeDtypeStruct((N, D), jnp.float32),
    )(x, W1)


def _preB_body(h_ref, hist_ref, hp_ref, dinv_ref):
    deg = hist_ref[0, :, 0:1] + hist_ref[1, :, 0:1] + 1.0
    dinv = lax.rsqrt(deg)
    hp = h_ref[...] * dinv
    hp_ref[0] = hp[:, :H]
    hp_ref[1] = hp[:, H:]
    dinv_ref[...] = dinv


def _tc_preB(h, hist):
    return pl.pallas_call(
        _preB_body,
        grid=(N // BN,),
        in_specs=[
            pl.BlockSpec((BN, D), lambda i: (i, 0)),
            pl.BlockSpec((NC, BN, 16), lambda i: (0, i, 0)),
        ],
        out_specs=[
            pl.BlockSpec((NC, BN, H), lambda i: (0, i, 0)),
            pl.BlockSpec((BN, 1), lambda i: (i, 0)),
        ],
        out_shape=[
            jax.ShapeDtypeStruct((NC, N, H), jnp.float32),
            jax.ShapeDtypeStruct((N, 1), jnp.float32),
        ],
    )(h, hist)


def _mid_body(agg_ref, dinv_ref, b_ref, w_ref, hp_ref):
    a = jnp.concatenate([agg_ref[0], agg_ref[1]], axis=-1)
    x2 = jnp.maximum(a * dinv_ref[...] + b_ref[...], 0.0)
    h2 = jnp.dot(x2, w_ref[...], preferred_element_type=jnp.float32)
    hp = h2 * dinv_ref[...]
    hp_ref[0] = hp[:, :H]
    hp_ref[1] = hp[:, H:]


def _tc_mid(agg, dinv, b1, W2):
    return pl.pallas_call(
        _mid_body,
        grid=(N // BN,),
        in_specs=[
            pl.BlockSpec((NC, BN, H), lambda i: (0, i, 0)),
            pl.BlockSpec((BN, 1), lambda i: (i, 0)),
            pl.BlockSpec((1, D), lambda i: (0, 0)),
            pl.BlockSpec((D, D), lambda i: (0, 0)),
        ],
        out_specs=pl.BlockSpec((NC, BN, H), lambda i: (0, i, 0)),
        out_shape=jax.ShapeDtypeStruct((NC, N, H), jnp.float32),
    )(agg, dinv, b1, W2)


def _fin_body(agg_ref, dinv_ref, b_ref, wfc_ref, bfc_ref, o_ref, m_scr):
    i = pl.program_id(0)
    a = jnp.concatenate([agg_ref[0], agg_ref[1]], axis=-1)
    x3 = jnp.maximum(a * dinv_ref[...] + b_ref[...], 0.0)
    bm = jnp.max(x3, axis=0, keepdims=True)

    @pl.when(i == 0)
    def _():
        m_scr[...] = bm

    @pl.when(i > 0)
    def _():
        m_scr[...] = jnp.maximum(m_scr[...], bm)

    @pl.when(i == pl.num_programs(0) - 1)
    def _():
        o_ref[...] = (jnp.dot(m_scr[...], wfc_ref[...],
                              preferred_element_type=jnp.float32) + bfc_ref[...])


def _tc_fin(agg, dinv, b2, Wfc, bfc):
    return pl.pallas_call(
        _fin_body,
        grid=(N // BN,),
        in_specs=[
            pl.BlockSpec((NC, BN, H), lambda i: (0, i, 0)),
            pl.BlockSpec((BN, 1), lambda i: (i, 0)),
            pl.BlockSpec((1, D), lambda i: (0, 0)),
            pl.BlockSpec((D, D // 16), lambda i: (0, 0)),
            pl.BlockSpec((1, D // 16), lambda i: (0, 0)),
        ],
        out_specs=pl.BlockSpec((1, D // 16), lambda i: (0, 0)),
        out_shape=jax.ShapeDtypeStruct((1, D // 16), jnp.float32),
        scratch_shapes=[pltpu.VMEM((1, D), jnp.float32)],
    )(agg, dinv, b2, Wfc, bfc)


def kernel(x, edge_index, W1, b1, W2, b2, Wfc, bfc):
    src = edge_index[0].astype(jnp.int32)
    dst = edge_index[1].astype(jnp.int32)
    pad = EPAD - E
    srcp = jnp.concatenate([src, jnp.zeros((pad,), jnp.int32)])
    dstp = jnp.concatenate([dst, jnp.full((pad,), N, jnp.int32)])
    src2 = jnp.concatenate([srcp, srcp + N])
    zeros = jnp.zeros((N, 16), jnp.float32)
    ones = jnp.concatenate(
        [jnp.ones((KH, 1), jnp.float32), jnp.zeros((KH, 15), jnp.float32)], axis=1)

    h1 = _tc_preA(x, W1)
    hist = _sc_hist(dstp, zeros, ones)
    hp1, dinv = _tc_preB(h1, hist)
    agg1 = _sc_agg(hp1.reshape(NC * N, H), src2, dstp)
    hp2 = _tc_mid(agg1, dinv, b1.reshape(1, D), W2)
    agg2 = _sc_agg(hp2.reshape(NC * N, H), src2, dstp)
    return _tc_fin(agg2, dinv, b2.reshape(1, D), Wfc, bfc.reshape(1, 16))


# speedup vs baseline: 1.0020x; 1.0020x over previous
import jax
import jax.numpy as jnp
from jax import lax
from jax.experimental import pallas as pl
from jax.experimental.pallas import tpu as pltpu
from jax.experimental.pallas import tpu_sc as plsc

N = 10000
E = 160000
D = 256
H = 128
NC = 2
NS = 16
NT0 = 624
NTL = N - NT0 * (NS - 1)

KA = 128
NBA = 80
ETILE = KA * NBA
EPAD = ETILE * NS
NSLOT = 2
NACC = N + 8

KH = 128
ETH = EPAD // (NC * NS)
NBH = ETH // KH

_MESH = plsc.VectorSubcoreMesh(core_axis_name="c", subcore_axis_name="s")


def _per_tile_rows(s, do):
    @pl.when(s < NS - 1)
    def _():
        do(s * NT0, NT0)

    @pl.when(s == NS - 1)
    def _():
        do((NS - 1) * NT0, NTL)


def _hist_body(dst_hbm, zeros_hbm, ones_hbm, out_hbm, onesv, dstring, cnt_sh,
               *sems):
    dsem = sems[:NSLOT]
    ssem = sems[NSLOT:]
    c = lax.axis_index("c")
    s = lax.axis_index("s")
    _per_tile_rows(s, lambda r, n: pltpu.sync_copy(
        zeros_hbm.at[pl.ds(r, n)], cnt_sh.at[pl.ds(r, n)]))
    pltpu.sync_copy(ones_hbm, onesv)
    plsc.subcore_barrier()
    base0 = (s * NC + c) * ETH

    def start_d(b, i):
        pltpu.async_copy(dst_hbm.at[pl.ds(base0 + i * KH, KH)], dstring.at[b],
                         dsem[b])

    def wait_d(b, i):
        pltpu.make_async_copy(dst_hbm.at[pl.ds(base0 + i * KH, KH)],
                              dstring.at[b], dsem[b]).wait()

    def start_s(b):
        pltpu.async_copy(onesv, cnt_sh.at[dstring.at[b]], ssem[b], add=True)

    def wait_s(b):
        pltpu.make_async_copy(onesv, cnt_sh.at[dstring.at[b]], ssem[b]).wait()

    for b in range(NSLOT):
        start_d(b, b)

    def outer(g, carry):
        for b in range(NSLOT):
            i = g * NSLOT + b
            wait_d(b, i)
            start_s(b)
            wait_s(b)

            @pl.when(i + NSLOT < NBH)
            def _():
                start_d(b, i + NSLOT)
        return carry

    lax.fori_loop(0, NBH // NSLOT, outer, 0)
    plsc.subcore_barrier()
    _per_tile_rows(s, lambda r, n: pltpu.sync_copy(
        cnt_sh.at[pl.ds(r, n)], out_hbm.at[c, pl.ds(r, n)]))


def _sc_hist(dstp_flat, zeros, ones):
    return pl.kernel(
        _hist_body,
        out_type=jax.ShapeDtypeStruct((NC, N, 16), jnp.float32),
        mesh=_MESH,
        scratch_types=[
            pltpu.VMEM((KH, 16), jnp.float32),
            pltpu.VMEM((NSLOT, KH), jnp.int32),
            pltpu.VMEM_SHARED((NACC, 16), jnp.float32),
        ] + [pltpu.SemaphoreType.DMA] * (2 * NSLOT),
    )(dstp_flat, zeros, ones)


def _agg_body(hp_hbm, src2_hbm, dst_hbm, out_hbm, src_all, dstring, rows,
              acc_sh, *sems):
    gsem = sems[:NSLOT]
    ssem = sems[NSLOT:2 * NSLOT]
    dsem = sems[2 * NSLOT:]
    c = lax.axis_index("c")
    s = lax.axis_index("s")
    pltpu.sync_copy(src2_hbm.at[pl.ds(c * EPAD + s * ETILE, ETILE)], src_all)
    _per_tile_rows(s, lambda r, n: pltpu.sync_copy(
        hp_hbm.at[pl.ds(c * N + r, n)], acc_sh.at[pl.ds(r, n)]))
    plsc.subcore_barrier()
    ebase = s * ETILE

    def start_d(b, i):
        pltpu.async_copy(dst_hbm.at[pl.ds(ebase + i * KA, KA)], dstring.at[b],
                         dsem[b])

    def wait_d(b, i):
        pltpu.make_async_copy(dst_hbm.at[pl.ds(ebase + i * KA, KA)],
                              dstring.at[b], dsem[b]).wait()

    def start_g(b, i):
        pltpu.async_copy(hp_hbm.at[src_all.at[pl.ds(i * KA, KA)]], rows.at[b],
                         gsem[b])

    def wait_g(b, i):
        pltpu.make_async_copy(hp_hbm.at[src_all.at[pl.ds(i * KA, KA)]],
                              rows.at[b], gsem[b]).wait()

    def start_s(b, i):
        pltpu.async_copy(rows.at[b], acc_sh.at[dstring.at[b]], ssem[b],
                         add=True)

    def wait_s(b, i):
        pltpu.make_async_copy(rows.at[b], acc_sh.at[dstring.at[b]],
                              ssem[b]).wait()

    for b in range(NSLOT):
        start_d(b, b)
        start_g(b, b)

    def outer(g, carry):
        for b in range(NSLOT):
            i = g * NSLOT + b
            wait_d(b, i)
            wait_g(b, i)
            start_s(b, i)
            wait_s(b, i)

            @pl.when(i + NSLOT < NBA)
            def _():
                start_d(b, i + NSLOT)
                start_g(b, i + NSLOT)
        return carry

    lax.fori_loop(0, NBA // NSLOT, outer, 0)
    plsc.subcore_barrier()
    _per_tile_rows(s, lambda r, n: pltpu.sync_copy(
        acc_sh.at[pl.ds(r, n)], out_hbm.at[c, pl.ds(r, n)]))


def _sc_agg(hp_flat, src2, dst):
    return pl.kernel(
        _agg_body,
        out_type=jax.ShapeDtypeStruct((NC, N, H), jnp.float32),
        mesh=_MESH,
        scratch_types=[
            pltpu.VMEM((ETILE,), jnp.int32),
            pltpu.VMEM((NSLOT, KA), jnp.int32),
            pltpu.VMEM((NSLOT, KA, H), jnp.float32),
            pltpu.VMEM_SHARED((NACC, H), jnp.float32),
        ] + [pltpu.SemaphoreType.DMA] * (3 * NSLOT),
    )(hp_flat, src2, dst)


BN = 1000


def _preA_body(x_ref, w_ref, h_ref):
    h_ref[...] = jnp.dot(x_ref[...], w_ref[...],
                         preferred_element_type=jnp.float32)


def _tc_preA(x, W1):
    return pl.pallas_call(
        _preA_body,
        grid=(N // BN,),
        in_specs=[
            pl.BlockSpec((BN, D), lambda i: (i, 0)),
            pl.BlockSpec((D, D), lambda i: (0, 0)),
        ],
        out_specs=pl.BlockSpec((BN, D), lambda i: (i, 0)),
        out_shape=jax.ShapeDtypeStruct((N, D), jnp.float32),
    )(x, W1)


def _preB_body(h_ref, hist_ref, hp_ref, dinv_ref):
    deg = hist_ref[0, :, 0:1] + hist_ref[1, :, 0:1] + 1.0
    dinv = lax.rsqrt(deg)
    hp = h_ref[...] * dinv
    hp_ref[0] = hp[:, :H]
    hp_ref[1] = hp[:, H:]
    dinv_ref[...] = dinv


def _tc_preB(h, hist):
    return pl.pallas_call(
        _preB_body,
        grid=(N // BN,),
        in_specs=[
            pl.BlockSpec((BN, D), lambda i: (i, 0)),
            pl.BlockSpec((NC, BN, 16), lambda i: (0, i, 0)),
        ],
        out_specs=[
            pl.BlockSpec((NC, BN, H), lambda i: (0, i, 0)),
            pl.BlockSpec((BN, 1), lambda i: (i, 0)),
        ],
        out_shape=[
            jax.ShapeDtypeStruct((NC, N, H), jnp.float32),
            jax.ShapeDtypeStruct((N, 1), jnp.float32),
        ],
    )(h, hist)


def _mid_body(agg_ref, dinv_ref, b_ref, w_ref, hp_ref):
    a = jnp.concatenate([agg_ref[0], agg_ref[1]], axis=-1)
    x2 = jnp.maximum(a * dinv_ref[...] + b_ref[...], 0.0)
    h2 = jnp.dot(x2, w_ref[...], preferred_element_type=jnp.float32)
    hp = h2 * dinv_ref[...]
    hp_ref[0] = hp[:, :H]
    hp_ref[1] = hp[:, H:]


def _tc_mid(agg, dinv, b1, W2):
    return pl.pallas_call(
        _mid_body,
        grid=(N // BN,),
        in_specs=[
            pl.BlockSpec((NC, BN, H), lambda i: (0, i, 0)),
            pl.BlockSpec((BN, 1), lambda i: (i, 0)),
            pl.BlockSpec((1, D), lambda i: (0, 0)),
            pl.BlockSpec((D, D), lambda i: (0, 0)),
        ],
        out_specs=pl.BlockSpec((NC, BN, H), lambda i: (0, i, 0)),
        out_shape=jax.ShapeDtypeStruct((NC, N, H), jnp.float32),
    )(agg, dinv, b1, W2)


def _fin_body(agg_ref, dinv_ref, b_ref, wfc_ref, bfc_ref, o_ref, m_scr):
    i = pl.program_id(0)
    a = jnp.concatenate([agg_ref[0], agg_ref[1]], axis=-1)
    x3 = jnp.maximum(a * dinv_ref[...] + b_ref[...], 0.0)
    bm = jnp.max(x3, axis=0, keepdims=True)

    @pl.when(i == 0)
    def _():
        m_scr[...] = bm

    @pl.when(i > 0)
    def _():
        m_scr[...] = jnp.maximum(m_scr[...], bm)

    @pl.when(i == pl.num_programs(0) - 1)
    def _():
        o_ref[...] = (jnp.dot(m_scr[...], wfc_ref[...],
                              preferred_element_type=jnp.float32) + bfc_ref[...])


def _tc_fin(agg, dinv, b2, Wfc, bfc):
    return pl.pallas_call(
        _fin_body,
        grid=(N // BN,),
        in_specs=[
            pl.BlockSpec((NC, BN, H), lambda i: (0, i, 0)),
            pl.BlockSpec((BN, 1), lambda i: (i, 0)),
            pl.BlockSpec((1, D), lambda i: (0, 0)),
            pl.BlockSpec((D, D // 16), lambda i: (0, 0)),
            pl.BlockSpec((1, D // 16), lambda i: (0, 0)),
        ],
        out_specs=pl.BlockSpec((1, D // 16), lambda i: (0, 0)),
        out_shape=jax.ShapeDtypeStruct((1, D // 16), jnp.float32),
        scratch_shapes=[pltpu.VMEM((1, D), jnp.float32)],
    )(agg, dinv, b2, Wfc, bfc)


def kernel(x, edge_index, W1, b1, W2, b2, Wfc, bfc):
    src = edge_index[0].astype(jnp.int32)
    dst = edge_index[1].astype(jnp.int32)
    pad = EPAD - E
    srcp = jnp.concatenate([src, jnp.zeros((pad,), jnp.int32)])
    dstp = jnp.concatenate([dst, jnp.full((pad,), N, jnp.int32)])
    src2 = jnp.concatenate([srcp, srcp + N])
    zeros = jnp.zeros((N, 16), jnp.float32)
    ones = jnp.concatenate(
        [jnp.ones((KH, 1), jnp.float32), jnp.zeros((KH, 15), jnp.float32)], axis=1)

    h1 = _tc_preA(x, W1)
    hist = _sc_hist(dstp, zeros, ones)
    hp1, dinv = _tc_preB(h1, hist)
    agg1 = _sc_agg(hp1.reshape(NC * N, H), src2, dstp)
    hp2 = _tc_mid(agg1, dinv, b1.reshape(1, D), W2)
    agg2 = _sc_agg(hp2.reshape(NC * N, H), src2, dstp)
    return _tc_fin(agg2, dinv, b2.reshape(1, D), Wfc, bfc.reshape(1, 16))
